# Initial kernel scaffold; baseline (speedup 1.0000x reference)
import jax
import jax.numpy as jnp
from jax.experimental import pallas as pl


def kernel(x, edge_index, W1, b1, W2, b2, W3, b3):
    raise NotImplementedError("write your pallas kernel here")


# speedup vs baseline: 5.5488x; 5.5488x over previous
import functools

import jax
import jax.numpy as jnp
from jax import lax
from jax.experimental import pallas as pl
from jax.experimental.pallas import tpu as pltpu
from jax.experimental.pallas import tpu_sc as plsc

N_NODES = 10000
N_PAD = 10240
D = 128
DH = 64
N_EDGES = 320000
NUM_CORES = 2
NUM_SUBCORES = 16
CHUNK = 128
NCH = -(-N_EDGES // (NUM_SUBCORES * CHUNK))
E_PAD = NUM_SUBCORES * CHUNK * NCH
ROWS_PER_SUB = N_PAD // NUM_SUBCORES
CW = 16


def _agg_body(with_counts, xv_hbm, si_hbm, di_hbm, *refs):
    if with_counts:
        (p_hbm, c_hbm, srcv, dstv, rows, zero16, onesv, acc, cnt, sem) = refs
    else:
        (p_hbm, srcv, dstv, rows, zero16, onesv, acc, cnt, sem) = refs
    c = lax.axis_index("c")
    s = lax.axis_index("s")

    pltpu.sync_copy(si_hbm.at[s], srcv)
    pltpu.sync_copy(di_hbm.at[s], dstv)

    @pl.loop(0, NCH)
    def _(r):
        @pl.loop(0, CHUNK // 16)
        def _(q):
            v = srcv[r, pl.ds(q * 16, 16)]
            srcv[r, pl.ds(q * 16, 16)] = v * 2 + c

    @pl.loop(0, CHUNK)
    def _(r):
        @pl.loop(0, DH // 16)
        def _(q):
            rows[r, pl.ds(q * 16, 16)] = jnp.zeros((16,), jnp.float32)

    if with_counts:
        @pl.loop(0, 64)
        def _(r):
            zero16[r, pl.ds(0, 16)] = jnp.zeros((16,), jnp.float32)

        @pl.loop(0, CHUNK)
        def _(r):
            onesv[r, pl.ds(0, 16)] = jnp.ones((16,), jnp.float32)

    base = s * ROWS_PER_SUB

    @pl.loop(0, ROWS_PER_SUB // CHUNK)
    def _(i):
        pltpu.sync_copy(rows, acc.at[pl.ds(base + i * CHUNK, CHUNK)])

    if with_counts:
        @pl.when(c == 0)
        def _():
            @pl.loop(0, ROWS_PER_SUB // 64)
            def _(i):
                pltpu.sync_copy(zero16, cnt.at[pl.ds(base + i * 64, 64)])

    plsc.subcore_barrier()

    @pl.loop(0, NCH)
    def _(j):
        pltpu.async_copy(xv_hbm.at[srcv.at[j]], rows, sem).wait()
        pltpu.sync_copy(rows, acc.at[dstv.at[j]], add=True)
        if with_counts:
            @pl.when(c == 0)
            def _():
                pltpu.sync_copy(onesv, cnt.at[dstv.at[j]], add=True)

    plsc.subcore_barrier()

    pltpu.sync_copy(acc.at[pl.ds(base, ROWS_PER_SUB)],
                    p_hbm.at[pl.ds(base, ROWS_PER_SUB), c])
    if with_counts:
        @pl.when(c == 0)
        def _():
            pltpu.sync_copy(cnt.at[pl.ds(base, ROWS_PER_SUB)],
                            c_hbm.at[pl.ds(base, ROWS_PER_SUB)])


@functools.cache
def _make_agg(n_src_rows, with_counts):
    mesh = plsc.VectorSubcoreMesh(core_axis_name="c", subcore_axis_name="s")
    out_type = [jax.ShapeDtypeStruct((N_PAD, NUM_CORES, DH), jnp.float32)]
    if with_counts:
        out_type.append(jax.ShapeDtypeStruct((N_PAD, CW), jnp.float32))
    scratch_types = [
        pltpu.VMEM((NCH, CHUNK), jnp.int32),
        pltpu.VMEM((NCH, CHUNK), jnp.int32),
        pltpu.VMEM((CHUNK, DH), jnp.float32),
        pltpu.VMEM((64, CW), jnp.float32),
        pltpu.VMEM((CHUNK, CW), jnp.float32),
        pltpu.VMEM_SHARED((N_PAD, DH), jnp.float32),
        pltpu.VMEM_SHARED((N_PAD, CW), jnp.float32),
        pltpu.SemaphoreType.DMA,
    ]
    return pl.kernel(functools.partial(_agg_body, with_counts),
                     out_type=out_type, mesh=mesh,
                     scratch_types=scratch_types,
                     compiler_params=pltpu.CompilerParams(
                         use_tc_tiling_on_sc=False))


def _update1_body(p_ref, c_ref, w_ref, b_ref, o_ref):
    ssum = p_ref[...]
    cnt = c_ref[:, 0:1]
    inv = 1.0 / jnp.maximum(cnt, 1.0)
    h = jnp.dot(ssum * inv, w_ref[...], preferred_element_type=jnp.float32)
    o_ref[...] = jnp.maximum(h + b_ref[...], 0.0)


def _update2_body(q_ref, c_ref, w2_ref, b2_ref, w3_ref, b3_ref, o_ref):
    ssum = q_ref[...]
    cnt = c_ref[:, 0:1]
    inv = 1.0 / jnp.maximum(cnt, 1.0)
    h = jnp.dot(ssum * inv, w2_ref[...], preferred_element_type=jnp.float32)
    h = jnp.maximum(h + b2_ref[...], 0.0)
    o_ref[...] = jnp.dot(h, w3_ref[...],
                         preferred_element_type=jnp.float32) + b3_ref[...]


_BLK = 512


def _update1(p, c, w1t, b1):
    return pl.pallas_call(
        _update1_body,
        grid=(N_PAD // _BLK,),
        in_specs=[
            pl.BlockSpec((_BLK, D), lambda i: (i, 0)),
            pl.BlockSpec((_BLK, CW), lambda i: (i, 0)),
            pl.BlockSpec((D, D), lambda i: (0, 0)),
            pl.BlockSpec((1, D), lambda i: (0, 0)),
        ],
        out_specs=pl.BlockSpec((_BLK, D), lambda i: (i, 0)),
        out_shape=jax.ShapeDtypeStruct((N_PAD, D), jnp.float32),
    )(p, c, w1t, b1)


def _update2(q, c, w2t, b2, w3t, b3):
    return pl.pallas_call(
        _update2_body,
        grid=(N_PAD // _BLK,),
        in_specs=[
            pl.BlockSpec((_BLK, D), lambda i: (i, 0)),
            pl.BlockSpec((_BLK, CW), lambda i: (i, 0)),
            pl.BlockSpec((D, D), lambda i: (0, 0)),
            pl.BlockSpec((1, D), lambda i: (0, 0)),
            pl.BlockSpec((D, D), lambda i: (0, 0)),
            pl.BlockSpec((1, D), lambda i: (0, 0)),
        ],
        out_specs=pl.BlockSpec((_BLK, D), lambda i: (i, 0)),
        out_shape=jax.ShapeDtypeStruct((N_PAD, D), jnp.float32),
    )(q, c, w2t, b2, w3t, b3)


def kernel(x, edge_index, W1, b1, W2, b2, W3, b3):
    src = edge_index[0].astype(jnp.int32)
    dst = edge_index[1].astype(jnp.int32)
    pad = E_PAD - N_EDGES
    srcp = jnp.concatenate([src, jnp.zeros((pad,), jnp.int32)])
    dstp = jnp.concatenate([dst, jnp.full((pad,), N_NODES, jnp.int32)])
    si = srcp.reshape(NUM_SUBCORES, NCH, CHUNK)
    di = dstp.reshape(NUM_SUBCORES, NCH, CHUNK)

    w1t = W1.T
    w2t = W2.T
    w3t = W3.T
    b1r = b1.reshape(1, D)
    b2r = b2.reshape(1, D)
    b3r = b3.reshape(1, D)

    xv = x.reshape(2 * N_NODES, DH)
    p1, c1 = _make_agg(2 * N_NODES, True)(xv, si, di)
    h1 = _update1(p1.reshape(N_PAD, D), c1, w1t, b1r)
    (p2,) = _make_agg(2 * N_PAD, False)(h1.reshape(2 * N_PAD, DH), si, di)
    out = _update2(p2.reshape(N_PAD, D), c1, w2t, b2r, w3t, b3r)
    return out[:N_NODES]


# speedup vs baseline: 7.2014x; 1.2978x over previous
import functools

import jax
import jax.numpy as jnp
from jax import lax
from jax.experimental import pallas as pl
from jax.experimental.pallas import tpu as pltpu
from jax.experimental.pallas import tpu_sc as plsc

N_NODES = 10000
N_PAD = 10240
D = 128
DH = 64
N_EDGES = 320000
NUM_CORES = 2
NUM_SUBCORES = 16
CHUNK = 125
NCH = N_EDGES // (NUM_SUBCORES * CHUNK)
EDGES_PER_SUB = NCH * CHUNK
ROWS_PER_SUB = N_PAD // NUM_SUBCORES
CW = 16


def _agg_body(with_counts, xv_hbm, si_hbm, di_hbm, *refs):
    if with_counts:
        (p_hbm, c_hbm, srcv, dstv, rows, zero16, onesv, acc, cnt, gsem) = refs
    else:
        (p_hbm, srcv, dstv, rows, zero16, onesv, acc, cnt, gsem) = refs
    c = lax.axis_index("c")
    s = lax.axis_index("s")

    pltpu.sync_copy(si_hbm.at[s], srcv)
    pltpu.sync_copy(di_hbm.at[s], dstv)

    @pl.loop(0, NCH)
    def _(r):
        vt = srcv[r, pl.ds(CHUNK - 16, 16)]
        for q in range(CHUNK // 16):
            v = srcv[r, pl.ds(q * 16, 16)]
            srcv[r, pl.ds(q * 16, 16)] = v * 2 + c
        srcv[r, pl.ds(CHUNK - 16, 16)] = vt * 2 + c

    @pl.loop(0, CHUNK)
    def _(r):
        for q in range(DH // 16):
            rows[0, r, pl.ds(q * 16, 16)] = jnp.zeros((16,), jnp.float32)

    if with_counts:
        @pl.loop(0, 64)
        def _(r):
            zero16[r, pl.ds(0, 16)] = jnp.zeros((16,), jnp.float32)

        @pl.loop(0, CHUNK)
        def _(r):
            onesv[r, pl.ds(0, 16)] = jnp.ones((16,), jnp.float32)

    base = s * ROWS_PER_SUB

    @pl.loop(0, ROWS_PER_SUB // 125)
    def _(i):
        pltpu.sync_copy(rows.at[0], acc.at[pl.ds(base + i * 125, 125)])
    pltpu.sync_copy(rows.at[0, pl.ds(0, 15)],
                    acc.at[pl.ds(base + 625, 15)])

    if with_counts:
        @pl.loop(0, ROWS_PER_SUB // 64)
        def _(i):
            pltpu.sync_copy(zero16, cnt.at[pl.ds(base + i * 64, 64)])

    plsc.subcore_barrier()

    def g_copy(k, b):
        return pltpu.make_async_copy(xv_hbm.at[srcv.at[k]], rows.at[b],
                                     gsem.at[b])

    g_copy(0, 0).start()

    @pl.loop(0, NCH, step=2)
    def _(j):
        for b in range(2):
            k = j + b
            g_copy(k, b).wait()

            @pl.when(k + 1 < NCH)
            def _():
                g_copy(k + 1, 1 - b).start()

            pltpu.sync_copy(rows.at[b], acc.at[dstv.at[k]], add=True)
            if with_counts:
                @pl.when((k < NCH // 2) == (c == 0))
                def _():
                    pltpu.sync_copy(onesv, cnt.at[dstv.at[k]], add=True)

    plsc.subcore_barrier()

    pltpu.sync_copy(acc.at[pl.ds(base, ROWS_PER_SUB)],
                    p_hbm.at[pl.ds(base, ROWS_PER_SUB), c])
    if with_counts:
        pltpu.sync_copy(cnt.at[pl.ds(base, ROWS_PER_SUB)],
                        c_hbm.at[pl.ds(base, ROWS_PER_SUB), c])


@functools.cache
def _make_agg(n_src_rows, with_counts):
    mesh = plsc.VectorSubcoreMesh(core_axis_name="c", subcore_axis_name="s")
    out_type = [jax.ShapeDtypeStruct((N_PAD, NUM_CORES, DH), jnp.float32)]
    if with_counts:
        out_type.append(
            jax.ShapeDtypeStruct((N_PAD, NUM_CORES, CW), jnp.float32))
    scratch_types = [
        pltpu.VMEM((NCH, CHUNK), jnp.int32),
        pltpu.VMEM((NCH, CHUNK), jnp.int32),
        pltpu.VMEM((2, CHUNK, DH), jnp.float32),
        pltpu.VMEM((64, CW), jnp.float32),
        pltpu.VMEM((CHUNK, CW), jnp.float32),
        pltpu.VMEM_SHARED((N_PAD, DH), jnp.float32),
        pltpu.VMEM_SHARED((N_PAD, CW), jnp.float32),
        pltpu.SemaphoreType.DMA((2,)),
    ]
    return pl.kernel(functools.partial(_agg_body, with_counts),
                     out_type=out_type, mesh=mesh,
                     scratch_types=scratch_types,
                     compiler_params=pltpu.CompilerParams(
                         use_tc_tiling_on_sc=False))


def _update1_body(p_ref, c_ref, w_ref, b_ref, o_ref):
    ssum = p_ref[...]
    cnt = c_ref[:, 0:1] + c_ref[:, CW:CW + 1]
    inv = 1.0 / jnp.maximum(cnt, 1.0)
    h = jnp.dot(ssum * inv, w_ref[...], preferred_element_type=jnp.float32)
    o_ref[...] = jnp.maximum(h + b_ref[...], 0.0)


def _update2_body(q_ref, c_ref, w2_ref, b2_ref, w3_ref, b3_ref, o_ref):
    ssum = q_ref[...]
    cnt = c_ref[:, 0:1] + c_ref[:, CW:CW + 1]
    inv = 1.0 / jnp.maximum(cnt, 1.0)
    h = jnp.dot(ssum * inv, w2_ref[...], preferred_element_type=jnp.float32)
    h = jnp.maximum(h + b2_ref[...], 0.0)
    o_ref[...] = jnp.dot(h, w3_ref[...],
                         preferred_element_type=jnp.float32) + b3_ref[...]


_BLK = 512


def _update1(p, c, w1t, b1):
    return pl.pallas_call(
        _update1_body,
        grid=(N_PAD // _BLK,),
        in_specs=[
            pl.BlockSpec((_BLK, D), lambda i: (i, 0)),
            pl.BlockSpec((_BLK, 2 * CW), lambda i: (i, 0)),
            pl.BlockSpec((D, D), lambda i: (0, 0)),
            pl.BlockSpec((1, D), lambda i: (0, 0)),
        ],
        out_specs=pl.BlockSpec((_BLK, D), lambda i: (i, 0)),
        out_shape=jax.ShapeDtypeStruct((N_PAD, D), jnp.float32),
    )(p, c, w1t, b1)


def _update2(q, c, w2t, b2, w3t, b3):
    return pl.pallas_call(
        _update2_body,
        grid=(N_PAD // _BLK,),
        in_specs=[
            pl.BlockSpec((_BLK, D), lambda i: (i, 0)),
            pl.BlockSpec((_BLK, 2 * CW), lambda i: (i, 0)),
            pl.BlockSpec((D, D), lambda i: (0, 0)),
            pl.BlockSpec((1, D), lambda i: (0, 0)),
            pl.BlockSpec((D, D), lambda i: (0, 0)),
            pl.BlockSpec((1, D), lambda i: (0, 0)),
        ],
        out_specs=pl.BlockSpec((_BLK, D), lambda i: (i, 0)),
        out_shape=jax.ShapeDtypeStruct((N_PAD, D), jnp.float32),
    )(q, c, w2t, b2, w3t, b3)


def kernel(x, edge_index, W1, b1, W2, b2, W3, b3):
    si = edge_index[0].astype(jnp.int32).reshape(NUM_SUBCORES, NCH, CHUNK)
    di = edge_index[1].astype(jnp.int32).reshape(NUM_SUBCORES, NCH, CHUNK)

    w1t = W1.T
    w2t = W2.T
    w3t = W3.T
    b1r = b1.reshape(1, D)
    b2r = b2.reshape(1, D)
    b3r = b3.reshape(1, D)

    xv = x.reshape(2 * N_NODES, DH)
    p1, c1 = _make_agg(2 * N_NODES, True)(xv, si, di)
    c1f = c1.reshape(N_PAD, NUM_CORES * CW)
    h1 = _update1(p1.reshape(N_PAD, D), c1f, w1t, b1r)
    (p2,) = _make_agg(2 * N_PAD, False)(h1.reshape(2 * N_PAD, DH), si, di)
    out = _update2(p2.reshape(N_PAD, D), c1f, w2t, b2r, w3t, b3r)
    return out[:N_NODES]


# speedup vs baseline: 7.4195x; 1.0303x over previous
import functools

import jax
import jax.numpy as jnp
from jax import lax
from jax.experimental import pallas as pl
from jax.experimental.pallas import tpu as pltpu
from jax.experimental.pallas import tpu_sc as plsc

N_NODES = 10000
N_PAD = 10240
D = 128
DH = 64
N_EDGES = 320000
NUM_CORES = 2
NUM_SUBCORES = 16
CHUNK = 125
NCH = N_EDGES // (NUM_SUBCORES * CHUNK)
EDGES_PER_SUB = NCH * CHUNK
ROWS_PER_SUB = N_PAD // NUM_SUBCORES
CW = 16


def _agg_body(with_counts, xv_hbm, si_hbm, di_hbm, *refs):
    if with_counts:
        (p_hbm, c_hbm, srcv, dstv, rows, zero16, onesv, acc, cnt, gsem) = refs
    else:
        (p_hbm, srcv, dstv, rows, zero16, onesv, acc, cnt, gsem) = refs
    c = lax.axis_index("c")
    s = lax.axis_index("s")

    pltpu.sync_copy(si_hbm.at[s], srcv)
    pltpu.sync_copy(di_hbm.at[s], dstv)

    @pl.loop(0, NCH)
    def _(r):
        vt = srcv[r, pl.ds(CHUNK - 16, 16)]
        for q in range(CHUNK // 16):
            v = srcv[r, pl.ds(q * 16, 16)]
            srcv[r, pl.ds(q * 16, 16)] = v * 2 + c
        srcv[r, pl.ds(CHUNK - 16, 16)] = vt * 2 + c

    @pl.loop(0, CHUNK)
    def _(r):
        for q in range(DH // 16):
            rows[0, r, pl.ds(q * 16, 16)] = jnp.zeros((16,), jnp.float32)

    if with_counts:
        @pl.loop(0, 64)
        def _(r):
            zero16[r, pl.ds(0, 16)] = jnp.zeros((16,), jnp.float32)

        @pl.loop(0, CHUNK)
        def _(r):
            onesv[r, pl.ds(0, 16)] = jnp.ones((16,), jnp.float32)

    base = s * ROWS_PER_SUB

    @pl.loop(0, ROWS_PER_SUB // 125)
    def _(i):
        pltpu.sync_copy(rows.at[0], acc.at[pl.ds(base + i * 125, 125)])
    pltpu.sync_copy(rows.at[0, pl.ds(0, 15)],
                    acc.at[pl.ds(base + 625, 15)])

    if with_counts:
        @pl.loop(0, ROWS_PER_SUB // 64)
        def _(i):
            pltpu.sync_copy(zero16, cnt.at[pl.ds(base + i * 64, 64)])

    plsc.subcore_barrier()

    def g_copy(k, b):
        return pltpu.make_async_copy(xv_hbm.at[srcv.at[k]], rows.at[b],
                                     gsem.at[b])

    g_copy(0, 0).start()

    @pl.loop(0, NCH, step=2)
    def _(j):
        for b in range(2):
            k = j + b
            g_copy(k, b).wait()

            @pl.when(k + 1 < NCH)
            def _():
                g_copy(k + 1, 1 - b).start()

            pltpu.sync_copy(rows.at[b], acc.at[dstv.at[k]], add=True)
            if with_counts:
                @pl.when((k < NCH // 2) == (c == 0))
                def _():
                    pltpu.sync_copy(onesv, cnt.at[dstv.at[k]], add=True)

    plsc.subcore_barrier()

    pltpu.sync_copy(acc.at[pl.ds(base, ROWS_PER_SUB)],
                    p_hbm.at[pl.ds(base, ROWS_PER_SUB), c])
    if with_counts:
        pltpu.sync_copy(cnt.at[pl.ds(base, ROWS_PER_SUB)],
                        c_hbm.at[pl.ds(base, ROWS_PER_SUB), c])


@functools.cache
def _make_agg(n_src_rows, with_counts):
    mesh = plsc.VectorSubcoreMesh(core_axis_name="c", subcore_axis_name="s")
    out_type = [jax.ShapeDtypeStruct((N_PAD, NUM_CORES, DH), jnp.float32)]
    if with_counts:
        out_type.append(
            jax.ShapeDtypeStruct((N_PAD, NUM_CORES, CW), jnp.float32))
    scratch_types = [
        pltpu.VMEM((NCH, CHUNK), jnp.int32),
        pltpu.VMEM((NCH, CHUNK), jnp.int32),
        pltpu.VMEM((2, CHUNK, DH), jnp.float32),
        pltpu.VMEM((64, CW), jnp.float32),
        pltpu.VMEM((CHUNK, CW), jnp.float32),
        pltpu.VMEM_SHARED((N_PAD, DH), jnp.float32),
        pltpu.VMEM_SHARED((N_PAD, CW), jnp.float32),
        pltpu.SemaphoreType.DMA((2,)),
    ]
    return pl.kernel(functools.partial(_agg_body, with_counts),
                     out_type=out_type, mesh=mesh,
                     scratch_types=scratch_types,
                     compiler_params=pltpu.CompilerParams(
                         use_tc_tiling_on_sc=False))


def _update1_body(p_ref, c_ref, w_ref, b_ref, o_ref):
    ssum = p_ref[...]
    cnt = c_ref[:, 0:1] + c_ref[:, CW:CW + 1]
    inv = 1.0 / jnp.maximum(cnt, 1.0)
    h = jnp.dot(ssum * inv, w_ref[...], preferred_element_type=jnp.float32)
    o_ref[...] = jnp.maximum(h + b_ref[...], 0.0)


def _update2_body(q_ref, c_ref, w2_ref, b2_ref, w3_ref, b3_ref, o_ref):
    ssum = q_ref[...]
    cnt = c_ref[:, 0:1] + c_ref[:, CW:CW + 1]
    inv = 1.0 / jnp.maximum(cnt, 1.0)
    h = jnp.dot(ssum * inv, w2_ref[...], preferred_element_type=jnp.float32)
    h = jnp.maximum(h + b2_ref[...], 0.0)
    o_ref[...] = jnp.dot(h, w3_ref[...],
                         preferred_element_type=jnp.float32) + b3_ref[...]


_BLK = 2048


def _update1(p, c, w1t, b1):
    return pl.pallas_call(
        _update1_body,
        grid=(N_PAD // _BLK,),
        in_specs=[
            pl.BlockSpec((_BLK, D), lambda i: (i, 0)),
            pl.BlockSpec((_BLK, 2 * CW), lambda i: (i, 0)),
            pl.BlockSpec((D, D), lambda i: (0, 0)),
            pl.BlockSpec((1, D), lambda i: (0, 0)),
        ],
        out_specs=pl.BlockSpec((_BLK, D), lambda i: (i, 0)),
        out_shape=jax.ShapeDtypeStruct((N_PAD, D), jnp.float32),
    )(p, c, w1t, b1)


def _update2(q, c, w2t, b2, w3t, b3):
    return pl.pallas_call(
        _update2_body,
        grid=(N_PAD // _BLK,),
        in_specs=[
            pl.BlockSpec((_BLK, D), lambda i: (i, 0)),
            pl.BlockSpec((_BLK, 2 * CW), lambda i: (i, 0)),
            pl.BlockSpec((D, D), lambda i: (0, 0)),
            pl.BlockSpec((1, D), lambda i: (0, 0)),
            pl.BlockSpec((D, D), lambda i: (0, 0)),
            pl.BlockSpec((1, D), lambda i: (0, 0)),
        ],
        out_specs=pl.BlockSpec((_BLK, D), lambda i: (i, 0)),
        out_shape=jax.ShapeDtypeStruct((N_PAD, D), jnp.float32),
    )(q, c, w2t, b2, w3t, b3)


def kernel(x, edge_index, W1, b1, W2, b2, W3, b3):
    si = edge_index[0].astype(jnp.int32).reshape(NUM_SUBCORES, NCH, CHUNK)
    di = edge_index[1].astype(jnp.int32).reshape(NUM_SUBCORES, NCH, CHUNK)

    w1t = W1.T
    w2t = W2.T
    w3t = W3.T
    b1r = b1.reshape(1, D)
    b2r = b2.reshape(1, D)
    b3r = b3.reshape(1, D)

    xv = x.reshape(2 * N_NODES, DH)
    p1, c1 = _make_agg(2 * N_NODES, True)(xv, si, di)
    c1f = c1.reshape(N_PAD, NUM_CORES * CW)
    h1 = _update1(p1.reshape(N_PAD, D), c1f, w1t, b1r)
    (p2,) = _make_agg(2 * N_PAD, False)(h1.reshape(2 * N_PAD, DH), si, di)
    out = _update2(p2.reshape(N_PAD, D), c1f, w2t, b2r, w3t, b3r)
    return out[:N_NODES]


# speedup vs baseline: 7.4885x; 1.0093x over previous
import functools

import jax
import jax.numpy as jnp
from jax import lax
from jax.experimental import pallas as pl
from jax.experimental.pallas import tpu as pltpu
from jax.experimental.pallas import tpu_sc as plsc

N_NODES = 10000
N_PAD = 10240
D = 128
DH = 64
N_EDGES = 320000
NUM_CORES = 2
NUM_SUBCORES = 16
CHUNK = 125
NCH = N_EDGES // (NUM_SUBCORES * CHUNK)
EDGES_PER_SUB = NCH * CHUNK
ROWS_PER_SUB = N_PAD // NUM_SUBCORES
CW = 16


def _agg_body(with_counts, xv_hbm, si_hbm, di_hbm, *refs):
    if with_counts:
        (p_hbm, c_hbm, srcv, dstv, rows, zero16, onesv, acc, cnt, gsem) = refs
    else:
        (p_hbm, srcv, dstv, rows, zero16, onesv, acc, cnt, gsem) = refs
    c = lax.axis_index("c")
    s = lax.axis_index("s")

    pltpu.sync_copy(si_hbm.at[s], srcv)
    pltpu.sync_copy(di_hbm.at[s], dstv)

    @pl.loop(0, NCH)
    def _(r):
        vt = srcv[r, pl.ds(CHUNK - 16, 16)]
        for q in range(CHUNK // 16):
            v = srcv[r, pl.ds(q * 16, 16)]
            srcv[r, pl.ds(q * 16, 16)] = v * 2 + c
        srcv[r, pl.ds(CHUNK - 16, 16)] = vt * 2 + c

    @pl.loop(0, CHUNK)
    def _(r):
        for q in range(DH // 16):
            rows[0, r, pl.ds(q * 16, 16)] = jnp.zeros((16,), jnp.float32)

    if with_counts:
        @pl.loop(0, 64)
        def _(r):
            zero16[r, pl.ds(0, 16)] = jnp.zeros((16,), jnp.float32)

        @pl.loop(0, CHUNK)
        def _(r):
            onesv[r, pl.ds(0, 16)] = jnp.ones((16,), jnp.float32)

    base = s * ROWS_PER_SUB

    @pl.loop(0, ROWS_PER_SUB // 125)
    def _(i):
        pltpu.sync_copy(rows.at[0], acc.at[pl.ds(base + i * 125, 125)])
    pltpu.sync_copy(rows.at[0, pl.ds(0, 15)],
                    acc.at[pl.ds(base + 625, 15)])

    if with_counts:
        @pl.loop(0, ROWS_PER_SUB // 64)
        def _(i):
            pltpu.sync_copy(zero16, cnt.at[pl.ds(base + i * 64, 64)])

    plsc.subcore_barrier()

    def g_copy(k, b):
        return pltpu.make_async_copy(xv_hbm.at[srcv.at[k]], rows.at[b],
                                     gsem.at[b])

    g_copy(0, 0).start()

    @pl.loop(0, NCH, step=2)
    def _(j):
        for b in range(2):
            k = j + b
            g_copy(k, b).wait()

            @pl.when(k + 1 < NCH)
            def _():
                g_copy(k + 1, 1 - b).start()

            pltpu.sync_copy(rows.at[b], acc.at[dstv.at[k]], add=True)
            if with_counts:
                @pl.when((k < NCH // 2) == (c == 0))
                def _():
                    pltpu.sync_copy(onesv, cnt.at[dstv.at[k]], add=True)

    plsc.subcore_barrier()

    pltpu.sync_copy(acc.at[pl.ds(base, ROWS_PER_SUB)],
                    p_hbm.at[pl.ds(base, ROWS_PER_SUB), c])
    if with_counts:
        pltpu.sync_copy(cnt.at[pl.ds(base, ROWS_PER_SUB)],
                        c_hbm.at[pl.ds(base, ROWS_PER_SUB), c])


@functools.cache
def _make_agg(n_src_rows, with_counts):
    mesh = plsc.VectorSubcoreMesh(core_axis_name="c", subcore_axis_name="s")
    out_type = [jax.ShapeDtypeStruct((N_PAD, NUM_CORES, DH), jnp.float32)]
    if with_counts:
        out_type.append(
            jax.ShapeDtypeStruct((N_PAD, NUM_CORES, CW), jnp.float32))
    scratch_types = [
        pltpu.VMEM((NCH, CHUNK), jnp.int32),
        pltpu.VMEM((NCH, CHUNK), jnp.int32),
        pltpu.VMEM((2, CHUNK, DH), jnp.float32),
        pltpu.VMEM((64, CW), jnp.float32),
        pltpu.VMEM((CHUNK, CW), jnp.float32),
        pltpu.VMEM_SHARED((N_PAD, DH), jnp.float32),
        pltpu.VMEM_SHARED((N_PAD, CW), jnp.float32),
        pltpu.SemaphoreType.DMA((2,)),
    ]
    return pl.kernel(functools.partial(_agg_body, with_counts),
                     out_type=out_type, mesh=mesh,
                     scratch_types=scratch_types,
                     compiler_params=pltpu.CompilerParams(
                         use_tc_tiling_on_sc=False))


def _update1_body(p_ref, c_ref, w_ref, b_ref, o_ref):
    ssum = p_ref[...]
    cnt = c_ref[:, 0:1] + c_ref[:, CW:CW + 1]
    inv = 1.0 / jnp.maximum(cnt, 1.0)
    h = jnp.dot(ssum * inv, w_ref[...], preferred_element_type=jnp.float32)
    o_ref[...] = jnp.maximum(h + b_ref[...], 0.0)


def _update2_body(q_ref, c_ref, w2_ref, b2_ref, w3_ref, b3_ref, o_ref):
    ssum = q_ref[:N_NODES, :]
    cnt = c_ref[:N_NODES, 0:1] + c_ref[:N_NODES, CW:CW + 1]
    inv = 1.0 / jnp.maximum(cnt, 1.0)
    h = jnp.dot(ssum * inv, w2_ref[...], preferred_element_type=jnp.float32)
    h = jnp.maximum(h + b2_ref[...], 0.0)
    o_ref[...] = jnp.dot(h, w3_ref[...],
                         preferred_element_type=jnp.float32) + b3_ref[...]


def _update1(p, c, w1t, b1):
    return pl.pallas_call(
        _update1_body,
        out_shape=jax.ShapeDtypeStruct((N_PAD, D), jnp.float32),
    )(p, c, w1t, b1)


def _update2(q, c, w2t, b2, w3t, b3):
    return pl.pallas_call(
        _update2_body,
        out_shape=jax.ShapeDtypeStruct((N_NODES, D), jnp.float32),
    )(q, c, w2t, b2, w3t, b3)


def kernel(x, edge_index, W1, b1, W2, b2, W3, b3):
    si = edge_index[0].astype(jnp.int32).reshape(NUM_SUBCORES, NCH, CHUNK)
    di = edge_index[1].astype(jnp.int32).reshape(NUM_SUBCORES, NCH, CHUNK)

    w1t = W1.T
    w2t = W2.T
    w3t = W3.T
    b1r = b1.reshape(1, D)
    b2r = b2.reshape(1, D)
    b3r = b3.reshape(1, D)

    xv = x.reshape(2 * N_NODES, DH)
    p1, c1 = _make_agg(2 * N_NODES, True)(xv, si, di)
    c1f = c1.reshape(N_PAD, NUM_CORES * CW)
    h1 = _update1(p1.reshape(N_PAD, D), c1f, w1t, b1r)
    (p2,) = _make_agg(2 * N_PAD, False)(h1.reshape(2 * N_PAD, DH), si, di)
    return _update2(p2.reshape(N_PAD, D), c1f, w2t, b2r, w3t, b3r)
